# Initial kernel scaffold; baseline (speedup 1.0000x reference)
import jax
import jax.numpy as jnp
from jax.experimental import pallas as pl


def kernel(bboxes, labels, segm_masks):
    raise NotImplementedError("write your pallas kernel here")


# speedup vs baseline: 7.1468x; 7.1468x over previous
import jax
import jax.numpy as jnp
from jax.experimental import pallas as pl
from jax.experimental.pallas import tpu as pltpu

N = 1000
NPAD = 1024
H = 512
W = 512
OVERLAP_THR = 0.5


def _sort_kernel(s_col_ref, s_row_ref, labels_col_ref, order_ref, labels_s_ref):
    s_col = s_col_ref[...]
    s_row = s_row_ref[...]
    labels_col = labels_col_ref[...]
    i_col = jax.lax.broadcasted_iota(jnp.int32, (NPAD, NPAD), 0)
    j_row = jax.lax.broadcasted_iota(jnp.int32, (NPAD, NPAD), 1)
    cmp = (s_row > s_col) | ((s_row == s_col) & (j_row < i_col))
    rank_col = cmp.astype(jnp.int32).sum(axis=1, keepdims=True)
    eq = (rank_col == j_row).astype(jnp.int32)
    order_ref[...] = (eq * i_col).sum(axis=0, keepdims=True)
    labels_s_ref[...] = (eq * labels_col).sum(axis=0, keepdims=True)


def _main_kernel(order_ref, labels_s_ref, mask_ref, id_map_ref, kept_ref,
                 inst_ref):
    i = pl.program_id(0)

    @pl.when(i == 0)
    def _init():
        id_map_ref[...] = jnp.zeros((H, W), jnp.int32)
        kept_ref[...] = jnp.full((8, 128), -1, jnp.int32)
        inst_ref[0] = 1

    mask = mask_ref[0]
    id_map = id_map_ref[...]
    used = id_map > 0
    area = jnp.sum(mask.astype(jnp.int32))
    inter = jnp.sum((mask & used).astype(jnp.int32))
    frac = inter.astype(jnp.float32) / (area.astype(jnp.float32) + 1e-05)
    skip = (area == 0) | (frac > OVERLAP_THR)

    label_i = labels_s_ref[0, i]
    kept_label = jnp.where(skip, jnp.int32(-1), label_i)
    row = jax.lax.broadcasted_iota(jnp.int32, (8, 128), 0)
    col = jax.lax.broadcasted_iota(jnp.int32, (8, 128), 1)
    onehot = (row == i // 128) & (col == i % 128)
    kept_ref[...] = jnp.where(onehot, kept_label, kept_ref[...])

    @pl.when(jnp.logical_not(skip))
    def _paint():
        inst = inst_ref[0]
        id_map_ref[...] = jnp.where(mask & jnp.logical_not(used), inst, id_map)
        inst_ref[0] = inst + 1


def _run(scores, labels, segm_masks, interpret=False):
    s_pad = jnp.full((NPAD,), -1.0, jnp.float32).at[:N].set(scores)
    l_pad = jnp.zeros((NPAD,), jnp.int32).at[:N].set(labels.astype(jnp.int32))

    order, labels_s = pl.pallas_call(
        _sort_kernel,
        out_shape=[
            jax.ShapeDtypeStruct((1, NPAD), jnp.int32),
            jax.ShapeDtypeStruct((1, NPAD), jnp.int32),
        ],
        interpret=interpret,
    )(s_pad.reshape(NPAD, 1), s_pad.reshape(1, NPAD), l_pad.reshape(NPAD, 1))

    grid_spec = pltpu.PrefetchScalarGridSpec(
        num_scalar_prefetch=2,
        grid=(N,),
        in_specs=[
            pl.BlockSpec((1, H, W), lambda i, order, labels_s: (order[0, i], 0, 0)),
        ],
        out_specs=[
            pl.BlockSpec((H, W), lambda i, order, labels_s: (0, 0)),
            pl.BlockSpec((8, 128), lambda i, order, labels_s: (0, 0)),
        ],
        scratch_shapes=[pltpu.SMEM((1,), jnp.int32)],
    )
    id_map, kept_pad = pl.pallas_call(
        _main_kernel,
        grid_spec=grid_spec,
        out_shape=[
            jax.ShapeDtypeStruct((H, W), jnp.int32),
            jax.ShapeDtypeStruct((8, 128), jnp.int32),
        ],
        interpret=interpret,
    )(order, labels_s, segm_masks)

    kept_labels = kept_pad.reshape(NPAD)[:N]
    return id_map, kept_labels


def kernel(bboxes, labels, segm_masks):
    scores = bboxes[:, -1]
    return _run(scores, labels, segm_masks)
